# baseline (speedup 1.0000x reference)
import functools

import jax
import jax.numpy as jnp
from jax import lax
from jax.experimental import pallas as pl
from jax.experimental.pallas import tpu as pltpu
from jax.experimental.pallas import tpu_sc as plsc

_N = 8 * 96 * 224 * 224
_NW = 32
_PER_W = _N // _NW
_CH = 24576
_NCH = _PER_W // _CH
_NV = _CH // 16
_UNROLL = 8


def _finalize(s_all, s_z, s_fn, c_z, c_tn, c_fn, c_tm):
    n_f = jnp.float32(_N)
    c_nz = _N - c_z
    s_nz = s_all - s_z
    c_tp = c_nz - c_fn
    s_tp = s_nz - s_fn
    c_fp = c_z - c_tn

    def mse(s, c, repl):
        m = s / jnp.maximum(c, 1).astype(jnp.float32)
        return jnp.where(c == 0, jnp.float32(repl), m)

    ff_loss = s_all / n_f
    zero_loss = mse(s_z, c_z, 0.0)
    nonzero_loss = mse(s_nz, c_nz, 0.0)
    time_match = jnp.where(c_tm == 0, jnp.float32(10.0), jnp.float32(0.0))
    fnl = mse(s_fn, c_fn, 0.0)
    fpl = mse(s_tp, c_tp, 0.0)
    tnl = jnp.where(c_tn == 0, jnp.float32(10.0), jnp.float32(0.0))
    tpl = mse(s_z, c_fp, 10.0)
    return (tpl + fnl + fpl + tnl + time_match
            + ff_loss + zero_loss + nonzero_loss)


def _sc_body(rec_hbm, tgt_hbm, out_hbm, bufr, buft, outbuf, sem0, sem1):
    wid = lax.axis_index("s") * 2 + lax.axis_index("c")
    base = wid * _PER_W

    def start(k, slot_r, slot_t, sem):
        pltpu.async_copy(rec_hbm.at[pl.ds(base + k * _CH, _CH)], slot_r, sem)
        pltpu.async_copy(tgt_hbm.at[pl.ds(base + k * _CH, _CH)], slot_t, sem)

    def drain(slot_r, slot_t, sem):
        pltpu.make_async_copy(rec_hbm.at[pl.ds(base, _CH)], slot_r, sem).wait()
        pltpu.make_async_copy(tgt_hbm.at[pl.ds(base, _CH)], slot_t, sem).wait()

    def chunk_acc(slot_r, slot_t, acc):
        def step(i, a):
            s_all, s_z, s_fn, c_z, c_rz, c_rt, c_tn = a
            for u in range(_UNROLL):
                off = (i * _UNROLL + u) * 16
                r = slot_r[pl.ds(off, 16)]
                t = slot_t[pl.ds(off, 16)]
                d = r - t
                sq = d * d
                zm = t == 0.0
                rz = r == 0.0
                rt = r == t
                tn = zm & rz
                fn = tn != rz
                zf = jnp.zeros((16,), jnp.float32)
                zi = jnp.zeros((16,), jnp.int32)
                oi = jnp.ones((16,), jnp.int32)
                s_all = s_all + sq
                s_z = s_z + jnp.where(zm, sq, zf)
                s_fn = s_fn + jnp.where(fn, sq, zf)
                c_z = c_z + jnp.where(zm, oi, zi)
                c_rz = c_rz + jnp.where(rz, oi, zi)
                c_rt = c_rt + jnp.where(rt, oi, zi)
                c_tn = c_tn + jnp.where(tn, oi, zi)
            return (s_all, s_z, s_fn, c_z, c_rz, c_rt, c_tn)
        return lax.fori_loop(0, _NV // _UNROLL, step, acc)

    acc0 = (tuple(jnp.zeros((16,), jnp.float32) for _ in range(3))
            + tuple(jnp.zeros((16,), jnp.int32) for _ in range(4)))

    start(0, bufr.at[0], buft.at[0], sem0)

    def outer(i, acc):
        k = i * 2
        start(k + 1, bufr.at[1], buft.at[1], sem1)
        drain(bufr.at[0], buft.at[0], sem0)
        acc = chunk_acc(bufr.at[0], buft.at[0], acc)
        start(k + 2, bufr.at[0], buft.at[0], sem0)
        drain(bufr.at[1], buft.at[1], sem1)
        return chunk_acc(bufr.at[1], buft.at[1], acc)

    acc = lax.fori_loop(0, (_NCH - 1) // 2, outer, acc0)
    drain(bufr.at[0], buft.at[0], sem0)
    acc = chunk_acc(bufr.at[0], buft.at[0], acc)

    for i in range(3):
        outbuf[i, :] = acc[i]
    for i in range(3, 7):
        outbuf[i, :] = acc[i].astype(jnp.float32)
    pltpu.sync_copy(outbuf, out_hbm.at[wid])


_sc_call = functools.partial(
    pl.kernel,
    out_type=jax.ShapeDtypeStruct((_NW, 7, 16), jnp.float32),
    mesh=plsc.VectorSubcoreMesh(core_axis_name="c", subcore_axis_name="s"),
    scratch_types=[
        pltpu.VMEM((2, _CH), jnp.float32),
        pltpu.VMEM((2, _CH), jnp.float32),
        pltpu.VMEM((7, 16), jnp.float32),
        pltpu.SemaphoreType.DMA,
        pltpu.SemaphoreType.DMA,
    ],
)(_sc_body)


def kernel(reconstructed_image, target_image):
    rec = reconstructed_image.reshape(_N)
    tgt = target_image.reshape(_N)
    parts = _sc_call(rec, tgt)
    s_all = jnp.sum(parts[:, 0, :])
    s_z = jnp.sum(parts[:, 1, :])
    s_fn = jnp.sum(parts[:, 2, :])
    counts = parts[:, 3:7, :].astype(jnp.int32)
    c_z = jnp.sum(counts[:, 0, :])
    c_rz = jnp.sum(counts[:, 1, :])
    c_rt = jnp.sum(counts[:, 2, :])
    c_tn = jnp.sum(counts[:, 3, :])
    c_fn = c_rz - c_tn
    c_tm = c_rt - c_tn
    return _finalize(s_all, s_z, s_fn, c_z, c_tn, c_fn, c_tm)


# speedup vs baseline: 1.0211x; 1.0211x over previous
import functools

import jax
import jax.numpy as jnp
from jax import lax
from jax.experimental import pallas as pl
from jax.experimental.pallas import tpu as pltpu
from jax.experimental.pallas import tpu_sc as plsc

_N = 8 * 96 * 224 * 224
_LANES = 1024
_ROWS = _N // _LANES

_NW = 32
_CH = 8192
_NV = _CH // 16
_UNROLL = 1
_SC_NCC = 57
_SC_PER_W = _SC_NCC * _CH
_SC_ELEMS = _NW * _SC_PER_W

_TC_BM = 768
_TC_OFF = _SC_ELEMS // (_LANES * _TC_BM)
_TC_GRID = _ROWS // _TC_BM - _TC_OFF
_SUB = 8


def _finalize(s_all, s_z, s_fn, c_z, c_tn, c_fn, c_tm):
    n_f = jnp.float32(_N)
    c_nz = _N - c_z
    s_nz = s_all - s_z
    c_tp = c_nz - c_fn
    s_tp = s_nz - s_fn
    c_fp = c_z - c_tn

    def mse(s, c, repl):
        m = s / jnp.maximum(c, 1).astype(jnp.float32)
        return jnp.where(c == 0, jnp.float32(repl), m)

    ff_loss = s_all / n_f
    zero_loss = mse(s_z, c_z, 0.0)
    nonzero_loss = mse(s_nz, c_nz, 0.0)
    time_match = jnp.where(c_tm == 0, jnp.float32(10.0), jnp.float32(0.0))
    fnl = mse(s_fn, c_fn, 0.0)
    fpl = mse(s_tp, c_tp, 0.0)
    tnl = jnp.where(c_tn == 0, jnp.float32(10.0), jnp.float32(0.0))
    tpl = mse(s_z, c_fp, 10.0)
    return (tpl + fnl + fpl + tnl + time_match
            + ff_loss + zero_loss + nonzero_loss)


def _sc_body(rec_hbm, tgt_hbm, out_hbm, bufr, buft, outbuf, sem0, sem1):
    wid = lax.axis_index("s") * 2 + lax.axis_index("c")
    base = wid * _SC_PER_W

    def start(k, slot_r, slot_t, sem):
        pltpu.async_copy(rec_hbm.at[pl.ds(base + k * _CH, _CH)], slot_r, sem)
        pltpu.async_copy(tgt_hbm.at[pl.ds(base + k * _CH, _CH)], slot_t, sem)

    def drain(slot_r, slot_t, sem):
        pltpu.make_async_copy(rec_hbm.at[pl.ds(base, _CH)], slot_r, sem).wait()
        pltpu.make_async_copy(tgt_hbm.at[pl.ds(base, _CH)], slot_t, sem).wait()

    def chunk_acc(slot_r, slot_t, acc):
        def step(i, a):
            s_all, s_z, s_fn, c_z, c_rz, c_rt, c_tn = a
            for u in range(_UNROLL):
                off = (i * _UNROLL + u) * 16
                r = slot_r[pl.ds(off, 16)]
                t = slot_t[pl.ds(off, 16)]
                d = r - t
                sq = d * d
                zm = t == 0.0
                rz = r == 0.0
                rt = r == t
                tn = zm & rz
                fn = tn != rz
                zf = jnp.zeros((16,), jnp.float32)
                zi = jnp.zeros((16,), jnp.int32)
                oi = jnp.ones((16,), jnp.int32)
                s_all = s_all + sq
                s_z = s_z + jnp.where(zm, sq, zf)
                s_fn = s_fn + jnp.where(fn, sq, zf)
                c_z = c_z + jnp.where(zm, oi, zi)
                c_rz = c_rz + jnp.where(rz, oi, zi)
                c_rt = c_rt + jnp.where(rt, oi, zi)
                c_tn = c_tn + jnp.where(tn, oi, zi)
            return (s_all, s_z, s_fn, c_z, c_rz, c_rt, c_tn)
        return lax.fori_loop(0, _NV // _UNROLL, step, acc)

    acc0 = (tuple(jnp.zeros((16,), jnp.float32) for _ in range(3))
            + tuple(jnp.zeros((16,), jnp.int32) for _ in range(4)))

    start(0, bufr.at[0], buft.at[0], sem0)

    def outer(i, acc):
        k = i * 2
        start(k + 1, bufr.at[1], buft.at[1], sem1)
        drain(bufr.at[0], buft.at[0], sem0)
        acc = chunk_acc(bufr.at[0], buft.at[0], acc)
        start(k + 2, bufr.at[0], buft.at[0], sem0)
        drain(bufr.at[1], buft.at[1], sem1)
        return chunk_acc(bufr.at[1], buft.at[1], acc)

    acc = lax.fori_loop(0, (_SC_NCC - 1) // 2, outer, acc0)
    drain(bufr.at[0], buft.at[0], sem0)
    acc = chunk_acc(bufr.at[0], buft.at[0], acc)

    for i in range(3):
        outbuf[i, :] = acc[i]
    for i in range(3, 7):
        outbuf[i, :] = acc[i].astype(jnp.float32)
    pltpu.sync_copy(outbuf, out_hbm.at[wid])


_sc_call_cache = []


def _sc_call(rec_flat, tgt_flat):
    if not _sc_call_cache:
        _sc_call_cache.append(functools.partial(
            pl.kernel,
            out_type=jax.ShapeDtypeStruct((_NW, 7, 16), jnp.float32),
            mesh=plsc.VectorSubcoreMesh(core_axis_name="c",
                                        subcore_axis_name="s"),
            scratch_types=[
                pltpu.VMEM((2, _CH), jnp.float32),
                pltpu.VMEM((2, _CH), jnp.float32),
                pltpu.VMEM((7, 16), jnp.float32),
                pltpu.SemaphoreType.DMA,
                pltpu.SemaphoreType.DMA,
            ],
        )(_sc_body))
    return _sc_call_cache[0](rec_flat, tgt_flat)


def _tc_body(rec_ref, tgt_ref, outf_ref, outi_ref, accf_ref, acci_ref):
    step = pl.program_id(0)

    @pl.when(step == 0)
    def _init():
        accf_ref[...] = jnp.zeros_like(accf_ref)
        acci_ref[...] = jnp.zeros_like(acci_ref)

    def fold(x):
        s = x[:, 0:128]
        for j in range(1, _LANES // 128):
            s = s + x[:, j * 128:(j + 1) * 128]
        return s

    for c in range(_TC_BM // _SUB):
        r = rec_ref[c * _SUB:(c + 1) * _SUB, :]
        t = tgt_ref[c * _SUB:(c + 1) * _SUB, :]
        d = r - t
        sq = d * d
        zm = t == 0.0
        rz = r == 0.0
        rt = r == t
        tn = zm & rz
        fn = tn != rz
        zero_f = jnp.zeros_like(sq)
        one_i = jnp.ones(sq.shape, jnp.int32)
        zero_i = jnp.zeros(sq.shape, jnp.int32)
        accf_ref[0] += fold(sq)
        accf_ref[1] += fold(jnp.where(zm, sq, zero_f))
        accf_ref[2] += fold(jnp.where(fn, sq, zero_f))
        acci_ref[0] += fold(jnp.where(zm, one_i, zero_i))
        acci_ref[1] += fold(jnp.where(rz, one_i, zero_i))
        acci_ref[2] += fold(jnp.where(rt, one_i, zero_i))
        acci_ref[3] += fold(jnp.where(tn, one_i, zero_i))

    @pl.when(step == _TC_GRID - 1)
    def _final():
        for i in range(3):
            outf_ref[0, i] = jnp.sum(accf_ref[i])
        for i in range(4):
            outi_ref[0, i] = jnp.sum(acci_ref[i])


def _tc_call(rec2d, tgt2d):
    return pl.pallas_call(
        _tc_body,
        grid=(_TC_GRID,),
        in_specs=[
            pl.BlockSpec((_TC_BM, _LANES), lambda i: (i + _TC_OFF, 0)),
            pl.BlockSpec((_TC_BM, _LANES), lambda i: (i + _TC_OFF, 0)),
        ],
        out_specs=[
            pl.BlockSpec(memory_space=pltpu.SMEM),
            pl.BlockSpec(memory_space=pltpu.SMEM),
        ],
        out_shape=[
            jax.ShapeDtypeStruct((1, 3), jnp.float32),
            jax.ShapeDtypeStruct((1, 4), jnp.int32),
        ],
        scratch_shapes=[
            pltpu.VMEM((3, _SUB, 128), jnp.float32),
            pltpu.VMEM((4, _SUB, 128), jnp.int32),
        ],
        compiler_params=pltpu.CompilerParams(
            dimension_semantics=("arbitrary",),
        ),
    )(rec2d, tgt2d)


def kernel(reconstructed_image, target_image):
    rec_flat = reconstructed_image.reshape(_N)
    tgt_flat = target_image.reshape(_N)
    rec2d = reconstructed_image.reshape(_ROWS, _LANES)
    tgt2d = target_image.reshape(_ROWS, _LANES)

    parts = _sc_call(rec_flat, tgt_flat)
    tf, ti = _tc_call(rec2d, tgt2d)

    s_all = jnp.sum(parts[:, 0, :]) + tf[0, 0]
    s_z = jnp.sum(parts[:, 1, :]) + tf[0, 1]
    s_fn = jnp.sum(parts[:, 2, :]) + tf[0, 2]
    counts = parts[:, 3:7, :].astype(jnp.int32)
    c_z = jnp.sum(counts[:, 0, :]) + ti[0, 0]
    c_rz = jnp.sum(counts[:, 1, :]) + ti[0, 1]
    c_rt = jnp.sum(counts[:, 2, :]) + ti[0, 2]
    c_tn = jnp.sum(counts[:, 3, :]) + ti[0, 3]
    c_fn = c_rz - c_tn
    c_tm = c_rt - c_tn
    return _finalize(s_all, s_z, s_fn, c_z, c_tn, c_fn, c_tm)


# speedup vs baseline: 1.1617x; 1.1377x over previous
import functools

import jax
import jax.numpy as jnp
from jax import lax
from jax.experimental import pallas as pl
from jax.experimental.pallas import tpu as pltpu
from jax.experimental.pallas import tpu_sc as plsc

_N = 8 * 96 * 224 * 224
_NW = 32
_CH = 8192
_NV = _CH // 16
_NCH = _N // (_NW * _CH)
_PER_W = _NCH * _CH


def _sc_body(rec_hbm, tgt_hbm, out_hbm, bufr, buft, outbuf, sem0, sem1):
    wid = lax.axis_index("s") * 2 + lax.axis_index("c")
    base = wid * _PER_W

    def start(k, slot_r, slot_t, sem):
        pltpu.async_copy(rec_hbm.at[pl.ds(base + k * _CH, _CH)], slot_r, sem)
        pltpu.async_copy(tgt_hbm.at[pl.ds(base + k * _CH, _CH)], slot_t, sem)

    def drain(slot_r, slot_t, sem):
        pltpu.make_async_copy(rec_hbm.at[pl.ds(base, _CH)], slot_r, sem).wait()
        pltpu.make_async_copy(tgt_hbm.at[pl.ds(base, _CH)], slot_t, sem).wait()

    def chunk_acc(slot_r, slot_t, acc):
        def step(i, a):
            s_all, s_z, s_fn, c_z, c_rz, c_rt, c_tn = a
            r = slot_r[pl.ds(i * 16, 16)]
            t = slot_t[pl.ds(i * 16, 16)]
            d = r - t
            sq = d * d
            zm = t == 0.0
            rz = r == 0.0
            rt = r == t
            tn = zm & rz
            fn = tn != rz
            zf = jnp.zeros((16,), jnp.float32)
            zi = jnp.zeros((16,), jnp.int32)
            oi = jnp.ones((16,), jnp.int32)
            return (s_all + sq,
                    s_z + jnp.where(zm, sq, zf),
                    s_fn + jnp.where(fn, sq, zf),
                    c_z + jnp.where(zm, oi, zi),
                    c_rz + jnp.where(rz, oi, zi),
                    c_rt + jnp.where(rt, oi, zi),
                    c_tn + jnp.where(tn, oi, zi))
        return lax.fori_loop(0, _NV, step, acc)

    acc0 = (tuple(jnp.zeros((16,), jnp.float32) for _ in range(3))
            + tuple(jnp.zeros((16,), jnp.int32) for _ in range(4)))

    start(0, bufr.at[0], buft.at[0], sem0)

    def outer(i, acc):
        k = i * 2
        start(k + 1, bufr.at[1], buft.at[1], sem1)
        drain(bufr.at[0], buft.at[0], sem0)
        acc = chunk_acc(bufr.at[0], buft.at[0], acc)
        start(k + 2, bufr.at[0], buft.at[0], sem0)
        drain(bufr.at[1], buft.at[1], sem1)
        return chunk_acc(bufr.at[1], buft.at[1], acc)

    acc = lax.fori_loop(0, (_NCH - 1) // 2, outer, acc0)
    drain(bufr.at[0], buft.at[0], sem0)
    acc = chunk_acc(bufr.at[0], buft.at[0], acc)

    for i in range(3):
        outbuf[i, :] = acc[i]
    for i in range(3, 7):
        outbuf[i, :] = acc[i].astype(jnp.float32)
    pltpu.sync_copy(outbuf, out_hbm.at[wid])


_sc_call_cache = []


def _sc_call(rec_flat, tgt_flat):
    if not _sc_call_cache:
        _sc_call_cache.append(functools.partial(
            pl.kernel,
            out_type=jax.ShapeDtypeStruct((_NW, 7, 16), jnp.float32),
            mesh=plsc.VectorSubcoreMesh(core_axis_name="c",
                                        subcore_axis_name="s"),
            scratch_types=[
                pltpu.VMEM((2, _CH), jnp.float32),
                pltpu.VMEM((2, _CH), jnp.float32),
                pltpu.VMEM((7, 16), jnp.float32),
                pltpu.SemaphoreType.DMA,
                pltpu.SemaphoreType.DMA,
            ],
        )(_sc_body))
    return _sc_call_cache[0](rec_flat, tgt_flat)


def _combine_body(parts_ref, out_ref):
    p = parts_ref[...]
    s_all = jnp.sum(p[:, 0, :])
    s_z = jnp.sum(p[:, 1, :])
    s_fn = jnp.sum(p[:, 2, :])
    c_z = jnp.sum(p[:, 3, :].astype(jnp.int32))
    c_rz = jnp.sum(p[:, 4, :].astype(jnp.int32))
    c_rt = jnp.sum(p[:, 5, :].astype(jnp.int32))
    c_tn = jnp.sum(p[:, 6, :].astype(jnp.int32))
    c_fn = c_rz - c_tn
    c_tm = c_rt - c_tn

    n_f = jnp.float32(_N)
    c_nz = _N - c_z
    s_nz = s_all - s_z
    c_tp = c_nz - c_fn
    s_tp = s_nz - s_fn
    c_fp = c_z - c_tn

    def mse(s, c, repl):
        m = s / jnp.maximum(c, 1).astype(jnp.float32)
        return jnp.where(c == 0, jnp.float32(repl), m)

    ff_loss = s_all / n_f
    zero_loss = mse(s_z, c_z, 0.0)
    nonzero_loss = mse(s_nz, c_nz, 0.0)
    time_match = jnp.where(c_tm == 0, jnp.float32(10.0), jnp.float32(0.0))
    fnl = mse(s_fn, c_fn, 0.0)
    fpl = mse(s_tp, c_tp, 0.0)
    tnl = jnp.where(c_tn == 0, jnp.float32(10.0), jnp.float32(0.0))
    tpl = mse(s_z, c_fp, 10.0)
    out_ref[0, 0] = (tpl + fnl + fpl + tnl + time_match
                     + ff_loss + zero_loss + nonzero_loss)


def _combine(parts, interpret=False):
    return pl.pallas_call(
        _combine_body,
        out_specs=pl.BlockSpec(memory_space=pltpu.SMEM),
        out_shape=jax.ShapeDtypeStruct((1, 1), jnp.float32),
        interpret=interpret,
    )(parts)


def kernel(reconstructed_image, target_image):
    rec_flat = reconstructed_image.reshape(_N)
    tgt_flat = target_image.reshape(_N)
    parts = _sc_call(rec_flat, tgt_flat)
    return _combine(parts)[0, 0]
